# baseline (speedup 1.0000x reference)
import functools

import jax
import jax.numpy as jnp
from jax import lax
from jax.experimental import pallas as pl
from jax.experimental.pallas import tpu as pltpu
from jax.experimental.pallas import tpu_sc as plsc

N = 10000
E = 320000
D = 128
G = 64

NC = 2
NS = 16
NW = NC * NS
W = 32
WPW = 320
CH = 40
NCHUNK = WPW // CH
NBUF = 8
EPW = W * WPW
E_PAD = NW * EPW
DUMP = N
ROWS_SP = NS * 632
ZROWS = 632

BN = 1000
NSTEPS = N // BN


def _sc_aggregate(z, srcp, dstp, zeros_hbm):
    mesh = plsc.VectorSubcoreMesh(core_axis_name="c", subcore_axis_name="s")

    @functools.partial(
        pl.kernel,
        out_type=[
            jax.ShapeDtypeStruct((N, D), jnp.float32),
            jax.ShapeDtypeStruct((N, D), jnp.float32),
        ],
        mesh=mesh,
        scratch_types=(
            [pltpu.VMEM((CH, W), jnp.int32),
             pltpu.VMEM((CH, W), jnp.int32)]
            + [pltpu.VMEM((W, D), jnp.float32)] * NBUF
            + [pltpu.VMEM_SHARED((ROWS_SP, D), jnp.float32)]
            + [pltpu.SemaphoreType.DMA] * (2 * NBUF)
        ),
    )
    def k(z_hbm, src_hbm, dst_hbm, zeros_ref, out0, out1, src_v, dst_v, *rest):
        bufs = rest[:NBUF]
        acc_sp = rest[NBUF]
        gsem = rest[NBUF + 1:2 * NBUF + 1]
        ssem = rest[2 * NBUF + 1:3 * NBUF + 1]
        c = lax.axis_index("c")
        s = lax.axis_index("s")
        wid = s * NC + c
        pltpu.sync_copy(zeros_ref, acc_sp.at[pl.ds(s * ZROWS, ZROWS)])
        plsc.subcore_barrier()

        def gather_start(j, b):
            pltpu.async_copy(z_hbm.at[src_v.at[j]], bufs[b], gsem[b])

        def gather_wait(j, b):
            pltpu.make_async_copy(z_hbm.at[src_v.at[j]], bufs[b], gsem[b]).wait()

        def scatter_start(j, b):
            pltpu.async_copy(bufs[b], acc_sp.at[dst_v.at[j]], ssem[b], add=True)

        def scatter_wait(j, b):
            pltpu.make_async_copy(bufs[b], acc_sp.at[dst_v.at[j]], ssem[b]).wait()

        for h in range(NCHUNK):
            base = wid * WPW + h * CH
            pltpu.sync_copy(src_hbm.at[pl.ds(base, CH)], src_v)
            pltpu.sync_copy(dst_hbm.at[pl.ds(base, CH)], dst_v)
            for b in range(NBUF):
                gather_start(b, b)

            @pl.loop(0, CH - NBUF, step=NBUF)
            def _(j):
                for b in range(NBUF):
                    gather_wait(j + b, b)
                    scatter_start(j + b, b)
                for b in range(NBUF):
                    scatter_wait(j + b, b)
                    gather_start(j + NBUF + b, b)

            for b in range(NBUF):
                gather_wait(CH - NBUF + b, b)
                scatter_start(CH - NBUF + b, b)
            for b in range(NBUF):
                scatter_wait(CH - NBUF + b, b)

        plsc.subcore_barrier()

        @pl.when(jnp.logical_and(c == 0, s < NS - 1))
        def _():
            pltpu.sync_copy(acc_sp.at[pl.ds(s * ZROWS, ZROWS)],
                            out0.at[pl.ds(s * ZROWS, ZROWS)])

        @pl.when(jnp.logical_and(c == 0, s == NS - 1))
        def _():
            pltpu.sync_copy(acc_sp.at[pl.ds((NS - 1) * ZROWS, N - (NS - 1) * ZROWS)],
                            out0.at[pl.ds((NS - 1) * ZROWS, N - (NS - 1) * ZROWS)])

        @pl.when(jnp.logical_and(c == 1, s < NS - 1))
        def _():
            pltpu.sync_copy(acc_sp.at[pl.ds(s * ZROWS, ZROWS)],
                            out1.at[pl.ds(s * ZROWS, ZROWS)])

        @pl.when(jnp.logical_and(c == 1, s == NS - 1))
        def _():
            pltpu.sync_copy(acc_sp.at[pl.ds((NS - 1) * ZROWS, N - (NS - 1) * ZROWS)],
                            out1.at[pl.ds((NS - 1) * ZROWS, N - (NS - 1) * ZROWS)])

    return k(z, srcp, dstp, zeros_hbm)


def _dot(a, b):
    return lax.dot_general(a, b, (((1,), (0,)), ((), ())),
                           preferred_element_type=jnp.float32,
                           precision=lax.Precision.HIGHEST)


def _mlp_body(z_ref, a0_ref, a1_ref, w1_ref, b1_ref, w2_ref, b2_ref, o_ref):
    h = z_ref[...] + a0_ref[...] + a1_ref[...]
    h = jnp.maximum(_dot(h, w1_ref[...]) + b1_ref[0:1, :], 0.0)
    h = _dot(h, w2_ref[...]) + b2_ref[0:1, :]
    o_ref[...] = jnp.maximum(h, 0.0)


def _mlp_pool_body(z_ref, a0_ref, a1_ref, w1_ref, b1_ref, w2_ref, b2_ref,
                   batch_ref, o_ref, g_ref, acc_ref):
    i = pl.program_id(0)

    @pl.when(i == 0)
    def _():
        acc_ref[...] = jnp.zeros_like(acc_ref)

    h = z_ref[...] + a0_ref[...] + a1_ref[...]
    h = jnp.maximum(_dot(h, w1_ref[...]) + b1_ref[0:1, :], 0.0)
    h = _dot(h, w2_ref[...]) + b2_ref[0:1, :]
    zo = jnp.maximum(h, 0.0)
    o_ref[...] = zo

    onehot = (batch_ref[...] ==
              lax.broadcasted_iota(jnp.int32, (BN, 128), 1)).astype(jnp.float32)
    ext = jnp.concatenate([zo, jnp.ones((BN, 128), jnp.float32)], axis=1)
    acc_ref[...] += lax.dot_general(onehot, ext, (((0,), (0,)), ((), ())),
                                    preferred_element_type=jnp.float32,
                                    precision=lax.Precision.HIGHEST)

    @pl.when(i == NSTEPS - 1)
    def _():
        sums = acc_ref[0:G, 0:128]
        cnts = acc_ref[0:G, 128:256]
        g_ref[...] = sums / jnp.maximum(cnts, 1.0)


_row_spec = pl.BlockSpec((BN, D), lambda i: (i, 0))
_mat_spec = pl.BlockSpec((D, D), lambda i: (0, 0))
_bias_spec = pl.BlockSpec((8, D), lambda i: (0, 0))


def _tc_mlp(z, a0, a1, W1, b1, W2, b2):
    return pl.pallas_call(
        _mlp_body,
        grid=(NSTEPS,),
        in_specs=[_row_spec, _row_spec, _row_spec,
                  _mat_spec, _bias_spec, _mat_spec, _bias_spec],
        out_specs=_row_spec,
        out_shape=jax.ShapeDtypeStruct((N, D), jnp.float32),
    )(z, a0, a1, W1, b1, W2, b2)


def _tc_mlp_pool(z, a0, a1, W1, b1, W2, b2, batch2d):
    return pl.pallas_call(
        _mlp_pool_body,
        grid=(NSTEPS,),
        in_specs=[_row_spec, _row_spec, _row_spec,
                  _mat_spec, _bias_spec, _mat_spec, _bias_spec,
                  pl.BlockSpec((BN, 1), lambda i: (i, 0))],
        out_specs=[_row_spec, pl.BlockSpec((G, D), lambda i: (0, 0))],
        out_shape=[jax.ShapeDtypeStruct((N, D), jnp.float32),
                   jax.ShapeDtypeStruct((G, D), jnp.float32)],
        scratch_shapes=[pltpu.VMEM((128, 256), jnp.float32)],
    )(z, a0, a1, W1, b1, W2, b2, batch2d)


def kernel(x, edge_index, batch, W1_0, b1_0, W2_0, b2_0,
           W1_1, b1_1, W2_1, b2_1, W1_2, b1_2, W2_2, b2_2):
    pad = E_PAD - E
    srcp = jnp.concatenate(
        [edge_index[0], jnp.arange(pad, dtype=jnp.int32) % N]).reshape(NW * WPW, W)
    dstp = jnp.concatenate(
        [edge_index[1],
         DUMP + (jnp.arange(pad, dtype=jnp.int32) % (ROWS_SP - N))]
    ).reshape(NW * WPW, W)
    zeros_hbm = jnp.zeros((ZROWS, D), jnp.float32)
    batch2d = batch.reshape(N, 1)
    btile = lambda b: jnp.broadcast_to(b.reshape(1, D), (8, D))

    params = [(W1_0, btile(b1_0), W2_0, btile(b2_0)),
              (W1_1, btile(b1_1), W2_1, btile(b2_1)),
              (W1_2, btile(b1_2), W2_2, btile(b2_2))]

    z = x
    for i, (W1, b1, W2, b2) in enumerate(params):
        agg0, agg1 = _sc_aggregate(z, srcp, dstp, zeros_hbm)
        if i < 2:
            z = _tc_mlp(z, agg0, agg1, W1, b1, W2, b2)
        else:
            z, g = _tc_mlp_pool(z, agg0, agg1, W1, b1, W2, b2, batch2d)
    return (z, g)


# speedup vs baseline: 1.0512x; 1.0512x over previous
import functools

import jax
import jax.numpy as jnp
from jax import lax
from jax.experimental import pallas as pl
from jax.experimental.pallas import tpu as pltpu
from jax.experimental.pallas import tpu_sc as plsc

N = 10000
E = 320000
D = 128
G = 64

NC = 2
NS = 16
NW = NC * NS
W = 64
WPW = 160
CH = 40
NCHUNK = WPW // CH
NBUF = 4
EPW = W * WPW
E_PAD = NW * EPW
DUMP = N
ROWS_SP = NS * 632
ZROWS = 632

BN = 1000
NSTEPS = N // BN


def _sc_aggregate(z, srcp, dstp, zeros_hbm):
    mesh = plsc.VectorSubcoreMesh(core_axis_name="c", subcore_axis_name="s")

    @functools.partial(
        pl.kernel,
        out_type=[
            jax.ShapeDtypeStruct((N, D), jnp.float32),
            jax.ShapeDtypeStruct((N, D), jnp.float32),
        ],
        mesh=mesh,
        scratch_types=(
            [pltpu.VMEM((CH, W), jnp.int32),
             pltpu.VMEM((CH, W), jnp.int32)]
            + [pltpu.VMEM((W, D), jnp.float32)] * NBUF
            + [pltpu.VMEM_SHARED((ROWS_SP, D), jnp.float32)]
            + [pltpu.SemaphoreType.DMA] * (2 * NBUF)
        ),
    )
    def k(z_hbm, src_hbm, dst_hbm, zeros_ref, out0, out1, src_v, dst_v, *rest):
        bufs = rest[:NBUF]
        acc_sp = rest[NBUF]
        gsem = rest[NBUF + 1:2 * NBUF + 1]
        ssem = rest[2 * NBUF + 1:3 * NBUF + 1]
        c = lax.axis_index("c")
        s = lax.axis_index("s")
        wid = s * NC + c
        pltpu.sync_copy(zeros_ref, acc_sp.at[pl.ds(s * ZROWS, ZROWS)])
        plsc.subcore_barrier()

        def gather_start(j, b):
            pltpu.async_copy(z_hbm.at[src_v.at[j]], bufs[b], gsem[b])

        def gather_wait(j, b):
            pltpu.make_async_copy(z_hbm.at[src_v.at[j]], bufs[b], gsem[b]).wait()

        def scatter_start(j, b):
            pltpu.async_copy(bufs[b], acc_sp.at[dst_v.at[j]], ssem[b], add=True)

        def scatter_wait(j, b):
            pltpu.make_async_copy(bufs[b], acc_sp.at[dst_v.at[j]], ssem[b]).wait()

        for h in range(NCHUNK):
            base = wid * WPW + h * CH
            pltpu.sync_copy(src_hbm.at[pl.ds(base, CH)], src_v)
            pltpu.sync_copy(dst_hbm.at[pl.ds(base, CH)], dst_v)
            for b in range(NBUF):
                gather_start(b, b)

            @pl.loop(0, CH - NBUF, step=NBUF)
            def _(j):
                for b in range(NBUF):
                    gather_wait(j + b, b)
                    scatter_start(j + b, b)
                for b in range(NBUF):
                    scatter_wait(j + b, b)
                    gather_start(j + NBUF + b, b)

            for b in range(NBUF):
                gather_wait(CH - NBUF + b, b)
                scatter_start(CH - NBUF + b, b)
            for b in range(NBUF):
                scatter_wait(CH - NBUF + b, b)

        plsc.subcore_barrier()

        @pl.when(jnp.logical_and(c == 0, s < NS - 1))
        def _():
            pltpu.sync_copy(acc_sp.at[pl.ds(s * ZROWS, ZROWS)],
                            out0.at[pl.ds(s * ZROWS, ZROWS)])

        @pl.when(jnp.logical_and(c == 0, s == NS - 1))
        def _():
            pltpu.sync_copy(acc_sp.at[pl.ds((NS - 1) * ZROWS, N - (NS - 1) * ZROWS)],
                            out0.at[pl.ds((NS - 1) * ZROWS, N - (NS - 1) * ZROWS)])

        @pl.when(jnp.logical_and(c == 1, s < NS - 1))
        def _():
            pltpu.sync_copy(acc_sp.at[pl.ds(s * ZROWS, ZROWS)],
                            out1.at[pl.ds(s * ZROWS, ZROWS)])

        @pl.when(jnp.logical_and(c == 1, s == NS - 1))
        def _():
            pltpu.sync_copy(acc_sp.at[pl.ds((NS - 1) * ZROWS, N - (NS - 1) * ZROWS)],
                            out1.at[pl.ds((NS - 1) * ZROWS, N - (NS - 1) * ZROWS)])

    return k(z, srcp, dstp, zeros_hbm)


def _dot(a, b):
    return lax.dot_general(a, b, (((1,), (0,)), ((), ())),
                           preferred_element_type=jnp.float32,
                           precision=lax.Precision.HIGHEST)


def _mlp_body(z_ref, a0_ref, a1_ref, w1_ref, b1_ref, w2_ref, b2_ref, o_ref):
    h = z_ref[...] + a0_ref[...] + a1_ref[...]
    h = jnp.maximum(_dot(h, w1_ref[...]) + b1_ref[0:1, :], 0.0)
    h = _dot(h, w2_ref[...]) + b2_ref[0:1, :]
    o_ref[...] = jnp.maximum(h, 0.0)


def _mlp_pool_body(z_ref, a0_ref, a1_ref, w1_ref, b1_ref, w2_ref, b2_ref,
                   batch_ref, o_ref, g_ref, acc_ref):
    i = pl.program_id(0)

    @pl.when(i == 0)
    def _():
        acc_ref[...] = jnp.zeros_like(acc_ref)

    h = z_ref[...] + a0_ref[...] + a1_ref[...]
    h = jnp.maximum(_dot(h, w1_ref[...]) + b1_ref[0:1, :], 0.0)
    h = _dot(h, w2_ref[...]) + b2_ref[0:1, :]
    zo = jnp.maximum(h, 0.0)
    o_ref[...] = zo

    onehot = (batch_ref[...] ==
              lax.broadcasted_iota(jnp.int32, (BN, 128), 1)).astype(jnp.float32)
    ext = jnp.concatenate([zo, jnp.ones((BN, 128), jnp.float32)], axis=1)
    acc_ref[...] += lax.dot_general(onehot, ext, (((0,), (0,)), ((), ())),
                                    preferred_element_type=jnp.float32,
                                    precision=lax.Precision.HIGHEST)

    @pl.when(i == NSTEPS - 1)
    def _():
        sums = acc_ref[0:G, 0:128]
        cnts = acc_ref[0:G, 128:256]
        g_ref[...] = sums / jnp.maximum(cnts, 1.0)


_row_spec = pl.BlockSpec((BN, D), lambda i: (i, 0))
_mat_spec = pl.BlockSpec((D, D), lambda i: (0, 0))
_bias_spec = pl.BlockSpec((8, D), lambda i: (0, 0))


def _tc_mlp(z, a0, a1, W1, b1, W2, b2):
    return pl.pallas_call(
        _mlp_body,
        grid=(NSTEPS,),
        in_specs=[_row_spec, _row_spec, _row_spec,
                  _mat_spec, _bias_spec, _mat_spec, _bias_spec],
        out_specs=_row_spec,
        out_shape=jax.ShapeDtypeStruct((N, D), jnp.float32),
    )(z, a0, a1, W1, b1, W2, b2)


def _tc_mlp_pool(z, a0, a1, W1, b1, W2, b2, batch2d):
    return pl.pallas_call(
        _mlp_pool_body,
        grid=(NSTEPS,),
        in_specs=[_row_spec, _row_spec, _row_spec,
                  _mat_spec, _bias_spec, _mat_spec, _bias_spec,
                  pl.BlockSpec((BN, 1), lambda i: (i, 0))],
        out_specs=[_row_spec, pl.BlockSpec((G, D), lambda i: (0, 0))],
        out_shape=[jax.ShapeDtypeStruct((N, D), jnp.float32),
                   jax.ShapeDtypeStruct((G, D), jnp.float32)],
        scratch_shapes=[pltpu.VMEM((128, 256), jnp.float32)],
    )(z, a0, a1, W1, b1, W2, b2, batch2d)


def kernel(x, edge_index, batch, W1_0, b1_0, W2_0, b2_0,
           W1_1, b1_1, W2_1, b2_1, W1_2, b1_2, W2_2, b2_2):
    pad = E_PAD - E
    srcp = jnp.concatenate(
        [edge_index[0], jnp.arange(pad, dtype=jnp.int32) % N]).reshape(NW * WPW, W)
    dstp = jnp.concatenate(
        [edge_index[1],
         DUMP + (jnp.arange(pad, dtype=jnp.int32) % (ROWS_SP - N))]
    ).reshape(NW * WPW, W)
    zeros_hbm = jnp.zeros((ZROWS, D), jnp.float32)
    batch2d = batch.reshape(N, 1)
    btile = lambda b: jnp.broadcast_to(b.reshape(1, D), (8, D))

    params = [(W1_0, btile(b1_0), W2_0, btile(b2_0)),
              (W1_1, btile(b1_1), W2_1, btile(b2_1)),
              (W1_2, btile(b1_2), W2_2, btile(b2_2))]

    z = x
    for i, (W1, b1, W2, b2) in enumerate(params):
        agg0, agg1 = _sc_aggregate(z, srcp, dstp, zeros_hbm)
        if i < 2:
            z = _tc_mlp(z, agg0, agg1, W1, b1, W2, b2)
        else:
            z, g = _tc_mlp_pool(z, agg0, agg1, W1, b1, W2, b2, batch2d)
    return (z, g)


# speedup vs baseline: 1.2063x; 1.1475x over previous
import functools

import jax
import jax.numpy as jnp
from jax import lax
from jax.experimental import pallas as pl
from jax.experimental.pallas import tpu as pltpu
from jax.experimental.pallas import tpu_sc as plsc

N = 10000
E = 320000
D = 128
G = 64

NC = 2
NS = 16
NW = NC * NS
W = 64
WPW = 160
CH = 40
NCHUNK = WPW // CH
NBUF = 4
EPW = W * WPW
E_PAD = NW * EPW
DUMP = N
ROWS_SP = NS * 632
ZROWS = 632

BN = 1000
NSTEPS = N // BN


def _sc_aggregate(z, srcp, dstp, zeros_hbm):
    mesh = plsc.VectorSubcoreMesh(core_axis_name="c", subcore_axis_name="s")

    @functools.partial(
        pl.kernel,
        out_type=[
            jax.ShapeDtypeStruct((N, D), jnp.float32),
            jax.ShapeDtypeStruct((N, D), jnp.float32),
        ],
        mesh=mesh,
        scratch_types=(
            [pltpu.VMEM((CH, W), jnp.int32),
             pltpu.VMEM((CH, W), jnp.int32)]
            + [pltpu.VMEM((W, D), jnp.float32)] * NBUF
            + [pltpu.VMEM_SHARED((ROWS_SP, D), jnp.float32)]
            + [pltpu.SemaphoreType.DMA] * (2 * NBUF)
        ),
    )
    def k(z_hbm, src_hbm, dst_hbm, zeros_ref, out0, out1, src_v, dst_v, *rest):
        bufs = rest[:NBUF]
        acc_sp = rest[NBUF]
        gsem = rest[NBUF + 1:2 * NBUF + 1]
        ssem = rest[2 * NBUF + 1:3 * NBUF + 1]
        c = lax.axis_index("c")
        s = lax.axis_index("s")
        wid = s * NC + c
        pltpu.sync_copy(zeros_ref, acc_sp.at[pl.ds(s * ZROWS, ZROWS)])
        plsc.subcore_barrier()

        def gather_start(j, b):
            pltpu.async_copy(z_hbm.at[src_v.at[j]], bufs[b], gsem[b])

        def gather_wait(j, b):
            pltpu.make_async_copy(z_hbm.at[src_v.at[j]], bufs[b], gsem[b]).wait()

        def scatter_start(j, b):
            pltpu.async_copy(bufs[b], acc_sp.at[dst_v.at[j]], ssem[b], add=True)

        def scatter_wait(j, b):
            pltpu.make_async_copy(bufs[b], acc_sp.at[dst_v.at[j]], ssem[b]).wait()

        for h in range(NCHUNK):
            base = wid * WPW + h * CH
            pltpu.sync_copy(src_hbm.at[pl.ds(base, CH)], src_v)
            pltpu.sync_copy(dst_hbm.at[pl.ds(base, CH)], dst_v)
            for b in range(NBUF):
                gather_start(b, b)

            @pl.loop(0, CH - NBUF, step=NBUF)
            def _(j):
                for b in range(NBUF):
                    gather_wait(j + b, b)
                    scatter_start(j + b, b)
                for b in range(NBUF):
                    scatter_wait(j + b, b)
                    gather_start(j + NBUF + b, b)

            for b in range(NBUF):
                gather_wait(CH - NBUF + b, b)
                scatter_start(CH - NBUF + b, b)
            for b in range(NBUF):
                scatter_wait(CH - NBUF + b, b)

        plsc.subcore_barrier()

        @pl.when(jnp.logical_and(c == 0, s < NS - 1))
        def _():
            pltpu.sync_copy(acc_sp.at[pl.ds(s * ZROWS, ZROWS)],
                            out0.at[pl.ds(s * ZROWS, ZROWS)])

        @pl.when(jnp.logical_and(c == 0, s == NS - 1))
        def _():
            pltpu.sync_copy(acc_sp.at[pl.ds((NS - 1) * ZROWS, N - (NS - 1) * ZROWS)],
                            out0.at[pl.ds((NS - 1) * ZROWS, N - (NS - 1) * ZROWS)])

        @pl.when(jnp.logical_and(c == 1, s < NS - 1))
        def _():
            pltpu.sync_copy(acc_sp.at[pl.ds(s * ZROWS, ZROWS)],
                            out1.at[pl.ds(s * ZROWS, ZROWS)])

        @pl.when(jnp.logical_and(c == 1, s == NS - 1))
        def _():
            pltpu.sync_copy(acc_sp.at[pl.ds((NS - 1) * ZROWS, N - (NS - 1) * ZROWS)],
                            out1.at[pl.ds((NS - 1) * ZROWS, N - (NS - 1) * ZROWS)])

    return k(z, srcp, dstp, zeros_hbm)


def _dot(a, b):
    return lax.dot_general(a, b, (((1,), (0,)), ((), ())),
                           preferred_element_type=jnp.float32)


def _mlp_body(z_ref, a0_ref, a1_ref, w1_ref, b1_ref, w2_ref, b2_ref, o_ref):
    h = z_ref[...] + a0_ref[...] + a1_ref[...]
    h = jnp.maximum(_dot(h, w1_ref[...]) + b1_ref[0:1, :], 0.0)
    h = _dot(h, w2_ref[...]) + b2_ref[0:1, :]
    o_ref[...] = jnp.maximum(h, 0.0)


def _mlp_pool_body(z_ref, a0_ref, a1_ref, w1_ref, b1_ref, w2_ref, b2_ref,
                   batch_ref, o_ref, g_ref, acc_ref):
    i = pl.program_id(0)

    @pl.when(i == 0)
    def _():
        acc_ref[...] = jnp.zeros_like(acc_ref)

    h = z_ref[...] + a0_ref[...] + a1_ref[...]
    h = jnp.maximum(_dot(h, w1_ref[...]) + b1_ref[0:1, :], 0.0)
    h = _dot(h, w2_ref[...]) + b2_ref[0:1, :]
    zo = jnp.maximum(h, 0.0)
    o_ref[...] = zo

    onehot = (batch_ref[...] ==
              lax.broadcasted_iota(jnp.int32, (BN, 128), 1)).astype(jnp.float32)
    ext = jnp.concatenate([zo, jnp.ones((BN, 128), jnp.float32)], axis=1)
    acc_ref[...] += lax.dot_general(onehot, ext, (((0,), (0,)), ((), ())),
                                    preferred_element_type=jnp.float32)

    @pl.when(i == NSTEPS - 1)
    def _():
        sums = acc_ref[0:G, 0:128]
        cnts = acc_ref[0:G, 128:256]
        g_ref[...] = sums / jnp.maximum(cnts, 1.0)


_row_spec = pl.BlockSpec((BN, D), lambda i: (i, 0))
_mat_spec = pl.BlockSpec((D, D), lambda i: (0, 0))
_bias_spec = pl.BlockSpec((8, D), lambda i: (0, 0))


def _tc_mlp(z, a0, a1, W1, b1, W2, b2):
    return pl.pallas_call(
        _mlp_body,
        grid=(NSTEPS,),
        in_specs=[_row_spec, _row_spec, _row_spec,
                  _mat_spec, _bias_spec, _mat_spec, _bias_spec],
        out_specs=_row_spec,
        out_shape=jax.ShapeDtypeStruct((N, D), jnp.float32),
    )(z, a0, a1, W1, b1, W2, b2)


def _tc_mlp_pool(z, a0, a1, W1, b1, W2, b2, batch2d):
    return pl.pallas_call(
        _mlp_pool_body,
        grid=(NSTEPS,),
        in_specs=[_row_spec, _row_spec, _row_spec,
                  _mat_spec, _bias_spec, _mat_spec, _bias_spec,
                  pl.BlockSpec((BN, 1), lambda i: (i, 0))],
        out_specs=[_row_spec, pl.BlockSpec((G, D), lambda i: (0, 0))],
        out_shape=[jax.ShapeDtypeStruct((N, D), jnp.float32),
                   jax.ShapeDtypeStruct((G, D), jnp.float32)],
        scratch_shapes=[pltpu.VMEM((128, 256), jnp.float32)],
    )(z, a0, a1, W1, b1, W2, b2, batch2d)


def kernel(x, edge_index, batch, W1_0, b1_0, W2_0, b2_0,
           W1_1, b1_1, W2_1, b2_1, W1_2, b1_2, W2_2, b2_2):
    pad = E_PAD - E
    srcp = jnp.concatenate(
        [edge_index[0], jnp.arange(pad, dtype=jnp.int32) % N]).reshape(NW * WPW, W)
    dstp = jnp.concatenate(
        [edge_index[1],
         DUMP + (jnp.arange(pad, dtype=jnp.int32) % (ROWS_SP - N))]
    ).reshape(NW * WPW, W)
    zeros_hbm = jnp.zeros((ZROWS, D), jnp.float32)
    batch2d = batch.reshape(N, 1)
    btile = lambda b: jnp.broadcast_to(b.reshape(1, D), (8, D))

    params = [(W1_0, btile(b1_0), W2_0, btile(b2_0)),
              (W1_1, btile(b1_1), W2_1, btile(b2_1)),
              (W1_2, btile(b1_2), W2_2, btile(b2_2))]

    z = x
    for i, (W1, b1, W2, b2) in enumerate(params):
        agg0, agg1 = _sc_aggregate(z, srcp, dstp, zeros_hbm)
        if i < 2:
            z = _tc_mlp(z, agg0, agg1, W1, b1, W2, b2)
        else:
            z, g = _tc_mlp_pool(z, agg0, agg1, W1, b1, W2, b2, batch2d)
    return (z, g)
